# baseline (speedup 1.0000x reference)
import functools

import jax
import jax.numpy as jnp
from jax import lax
from jax.experimental import pallas as pl
from jax.experimental.pallas import tpu as pltpu
from jax.experimental.pallas import tpu_sc as plsc

N_NODES = 10000
N_EDGES = 160000
D = 256
HALF = 128
L = 3
N_GRAPHS = 64
PH = 256
BN_EPS = 1e-5

NC = 2
NS = 16
EDGES_PER_SUB = N_EDGES // NS
CH = 80
NITER = EDGES_PER_SUB // CH
ROWS_A = 624
ROWS_LAST = N_NODES - (NS - 1) * ROWS_A

BLK = 1000
GRID = N_NODES // BLK


def _sc_agg_body(hcat, srcx, dst, zeros, agg,
                 sidx0, sidx1, didx0, didx1, rows0, rows1, acc, sg0, sg1):
    c = lax.axis_index("c")
    s = lax.axis_index("s")
    sidx = (sidx0, sidx1)
    didx = (didx0, didx1)
    rows = (rows0, rows1)
    sg = (sg0, sg1)

    @pl.when(s < NS - 1)
    def _():
        pltpu.sync_copy(zeros.at[pl.ds(s * ROWS_A, ROWS_A)],
                        acc.at[pl.ds(s * ROWS_A, ROWS_A)])

    @pl.when(s == NS - 1)
    def _():
        pltpu.sync_copy(zeros.at[pl.ds((NS - 1) * ROWS_A, ROWS_LAST)],
                        acc.at[pl.ds((NS - 1) * ROWS_A, ROWS_LAST)])

    plsc.subcore_barrier()

    ebase = c * N_EDGES + s * EDGES_PER_SUB
    dbase = s * EDGES_PER_SUB

    def load_idx(b, i):
        pltpu.sync_copy(srcx.at[pl.ds(ebase + i * CH, CH)], sidx[b])
        pltpu.sync_copy(dst.at[pl.ds(dbase + i * CH, CH)], didx[b])

    def start_gather(b):
        pltpu.async_copy(hcat.at[sidx[b]], rows[b], sg[b])

    def wait_gather(b):
        pltpu.make_async_copy(hcat.at[sidx[b]], rows[b], sg[b]).wait()

    def scatter(b):
        pltpu.sync_copy(rows[b], acc.at[didx[b]], add=True)

    load_idx(0, 0)
    start_gather(0)

    def pair(j, _):
        for b in range(2):
            i = 2 * j + b
            load_idx(1 - b, i + 1)
            start_gather(1 - b)

            wait_gather(b)
            scatter(b)
        return ()

    lax.fori_loop(0, (NITER - 1) // 2, pair, (), unroll=False)
    wait_gather(0)
    scatter(0)
    plsc.subcore_barrier()

    @pl.when(s < NS - 1)
    def _():
        pltpu.sync_copy(acc.at[pl.ds(s * ROWS_A, ROWS_A)],
                        agg.at[pl.ds(c * N_NODES + s * ROWS_A, ROWS_A)])

    @pl.when(s == NS - 1)
    def _():
        pltpu.sync_copy(
            acc.at[pl.ds((NS - 1) * ROWS_A, ROWS_LAST)],
            agg.at[pl.ds(c * N_NODES + (NS - 1) * ROWS_A, ROWS_LAST)])


@functools.cache
def _sc_agg():
    return pl.kernel(
        _sc_agg_body,
        out_type=jax.ShapeDtypeStruct((2 * N_NODES, HALF), jnp.float32),
        mesh=plsc.VectorSubcoreMesh(core_axis_name="c", subcore_axis_name="s",
                                    num_cores=NC, num_subcores=NS),
        scratch_types=[
            pltpu.VMEM((CH,), jnp.int32),
            pltpu.VMEM((CH,), jnp.int32),
            pltpu.VMEM((CH,), jnp.int32),
            pltpu.VMEM((CH,), jnp.int32),
            pltpu.VMEM((CH, HALF), jnp.float32),
            pltpu.VMEM((CH, HALF), jnp.float32),
            pltpu.VMEM_SHARED((N_NODES, HALF), jnp.float32),
            pltpu.SemaphoreType.DMA,
            pltpu.SemaphoreType.DMA,
        ],
    )


def _k1_body(eps_ref, hcat_ref, agg_ref, w_ref, b_ref, y_ref, st_ref):
    i = pl.program_id(0)

    @pl.when(i == 0)
    def _():
        st_ref[...] = jnp.zeros_like(st_ref)

    scale = 1.0 + eps_ref[0, 0]
    z = jnp.concatenate(
        [scale * hcat_ref[q] + agg_ref[q] for q in range(2)], axis=1)
    y = jnp.dot(z, w_ref[...], preferred_element_type=jnp.float32) + b_ref[...]
    y_ref[...] = y
    st_ref[...] += jnp.stack([jnp.sum(y, axis=0), jnp.sum(y * y, axis=0)])


def _bn_from_stats(st):
    m = st[0:1, :] / N_NODES
    v = st[1:2, :] / N_NODES - m * m
    return m, lax.rsqrt(v + BN_EPS)


def _k2_body(y_ref, st_ref, g_ref, be_ref, w_ref, b_ref, y2_ref, st2_ref):
    i = pl.program_id(0)

    @pl.when(i == 0)
    def _():
        st2_ref[...] = jnp.zeros_like(st2_ref)

    m, r = _bn_from_stats(st_ref[...])
    yn = jnp.maximum((y_ref[...] - m) * r * g_ref[...] + be_ref[...], 0.0)
    y2 = jnp.dot(yn, w_ref[...], preferred_element_type=jnp.float32) + b_ref[...]
    y2_ref[...] = y2
    st2_ref[...] += jnp.stack([jnp.sum(y2, axis=0), jnp.sum(y2 * y2, axis=0)])


def _k3_body(y_ref, st_ref, g_ref, be_ref, batch_ref, out_ref, pool_ref):
    i = pl.program_id(0)

    @pl.when(i == 0)
    def _():
        pool_ref[...] = jnp.zeros_like(pool_ref)

    m, r = _bn_from_stats(st_ref[...])
    h = jnp.maximum((y_ref[...] - m) * r * g_ref[...] + be_ref[...], 0.0)
    for q in range(2):
        out_ref[q] = h[:, HALF * q:HALF * (q + 1)]
    gids = lax.broadcasted_iota(jnp.int32, (BLK, N_GRAPHS), 1)
    mask = (batch_ref[...] == gids).astype(jnp.float32)
    pool_ref[...] += lax.dot_general(
        mask, h, (((0,), (0,)), ((), ())), preferred_element_type=jnp.float32,
        precision=lax.Precision.HIGHEST)


def _kfin_body(p1_ref, p2_ref, p3_ref, wd1_ref, bd1_ref, wd2_ref, bd2_ref,
               out_ref):
    cat = jnp.concatenate([p1_ref[...], p2_ref[...], p3_ref[...]], axis=1)
    t = jnp.maximum(
        jnp.dot(cat, wd1_ref[...], preferred_element_type=jnp.float32)
        + bd1_ref[...], 0.0)
    out_ref[...] = (jnp.dot(t, wd2_ref[...], preferred_element_type=jnp.float32)
                    + bd2_ref[0, 0])


_row_blk = pl.BlockSpec((BLK, D), lambda i: (i, 0))
_half_blk = pl.BlockSpec((2, BLK, HALF), lambda i: (0, i, 0))
_full_st = pl.BlockSpec((2, D), lambda i: (0, 0))
_full_vec = pl.BlockSpec((1, D), lambda i: (0, 0))
_full_w = pl.BlockSpec((D, D), lambda i: (0, 0))

_k1 = pl.pallas_call(
    _k1_body,
    grid=(GRID,),
    in_specs=[
        pl.BlockSpec(memory_space=pltpu.SMEM),
        _half_blk, _half_blk, _full_w, _full_vec,
    ],
    out_specs=[_row_blk, _full_st],
    out_shape=[
        jax.ShapeDtypeStruct((N_NODES, D), jnp.float32),
        jax.ShapeDtypeStruct((2, D), jnp.float32),
    ],
)

_k2 = pl.pallas_call(
    _k2_body,
    grid=(GRID,),
    in_specs=[_row_blk, _full_st, _full_vec, _full_vec, _full_w, _full_vec],
    out_specs=[_row_blk, _full_st],
    out_shape=[
        jax.ShapeDtypeStruct((N_NODES, D), jnp.float32),
        jax.ShapeDtypeStruct((2, D), jnp.float32),
    ],
)

_k3 = pl.pallas_call(
    _k3_body,
    grid=(GRID,),
    in_specs=[
        _row_blk, _full_st, _full_vec, _full_vec,
        pl.BlockSpec((BLK, 1), lambda i: (i, 0)),
    ],
    out_specs=[_half_blk, pl.BlockSpec((N_GRAPHS, D), lambda i: (0, 0))],
    out_shape=[
        jax.ShapeDtypeStruct((2, N_NODES, HALF), jnp.float32),
        jax.ShapeDtypeStruct((N_GRAPHS, D), jnp.float32),
    ],
)

_kfin = pl.pallas_call(
    _kfin_body,
    in_specs=[
        pl.BlockSpec((N_GRAPHS, D), lambda: (0, 0)),
        pl.BlockSpec((N_GRAPHS, D), lambda: (0, 0)),
        pl.BlockSpec((N_GRAPHS, D), lambda: (0, 0)),
        pl.BlockSpec((L * D, PH), lambda: (0, 0)),
        pl.BlockSpec((1, PH), lambda: (0, 0)),
        pl.BlockSpec((PH, 1), lambda: (0, 0)),
        pl.BlockSpec(memory_space=pltpu.SMEM),
    ],
    out_specs=pl.BlockSpec((N_GRAPHS, 1), lambda: (0, 0)),
    out_shape=jax.ShapeDtypeStruct((N_GRAPHS, 1), jnp.float32),
)


def kernel(x, edge_index, batch, eps, W1, b1, g1, be1, W2, b2, g2, be2,
           Wd1, bd1, Wd2, bd2):
    order = jnp.argsort(edge_index[1], stable=True)
    src = edge_index[0][order]
    dst = edge_index[1][order]
    srcx = jnp.concatenate([src, src + N_NODES])
    zeros = jnp.zeros((N_NODES, HALF), jnp.float32)
    batch2 = batch.reshape(N_NODES, 1)

    hcat = jnp.concatenate([x[:, :HALF], x[:, HALF:]], axis=0)

    pools = []
    for l in range(L):
        agg = _sc_agg()(hcat, srcx, dst, zeros)
        eps_l = eps[l].reshape(1, 1)
        h3 = hcat.reshape(2, N_NODES, HALF)
        a3 = agg.reshape(2, N_NODES, HALF)
        y1, st1 = _k1(eps_l, h3, a3, W1[l], b1[l].reshape(1, D))
        y2, st2 = _k2(y1, st1, g1[l].reshape(1, D), be1[l].reshape(1, D),
                      W2[l], b2[l].reshape(1, D))
        hnew, pool = _k3(y2, st2, g2[l].reshape(1, D), be2[l].reshape(1, D),
                         batch2)
        pools.append(pool)
        hcat = hnew.reshape(2 * N_NODES, HALF)

    out = _kfin(pools[0], pools[1], pools[2], Wd1, bd1.reshape(1, PH),
                Wd2, bd2.reshape(1, 1))
    return out


# speedup vs baseline: 1.3208x; 1.3208x over previous
import functools

import jax
import jax.numpy as jnp
from jax import lax
from jax.experimental import pallas as pl
from jax.experimental.pallas import tpu as pltpu
from jax.experimental.pallas import tpu_sc as plsc

N_NODES = 10000
N_EDGES = 160000
D = 256
HALF = 128
L = 3
N_GRAPHS = 64
PH = 256
BN_EPS = 1e-5

NC = 2
NS = 16
EDGES_PER_SUB = N_EDGES // NS
CH = 80
NITER = EDGES_PER_SUB // CH
ROWS_A = 624
ROWS_LAST = N_NODES - (NS - 1) * ROWS_A

BLK = 1000
GRID = N_NODES // BLK


def _sc_agg_body(hcat, srcx, dst, zeros, agg,
                 sidx0, sidx1, didx0, didx1, rows0, rows1, acc, sg0, sg1):
    c = lax.axis_index("c")
    s = lax.axis_index("s")
    sidx = (sidx0, sidx1)
    didx = (didx0, didx1)
    rows = (rows0, rows1)
    sg = (sg0, sg1)

    @pl.when(s < NS - 1)
    def _():
        pltpu.sync_copy(zeros.at[pl.ds(s * ROWS_A, ROWS_A)],
                        acc.at[pl.ds(s * ROWS_A, ROWS_A)])

    @pl.when(s == NS - 1)
    def _():
        pltpu.sync_copy(zeros.at[pl.ds((NS - 1) * ROWS_A, ROWS_LAST)],
                        acc.at[pl.ds((NS - 1) * ROWS_A, ROWS_LAST)])

    plsc.subcore_barrier()

    ebase = c * N_EDGES + s * EDGES_PER_SUB
    dbase = s * EDGES_PER_SUB

    def load_idx(b, i):
        pltpu.sync_copy(srcx.at[pl.ds(ebase + i * CH, CH)], sidx[b])
        pltpu.sync_copy(dst.at[pl.ds(dbase + i * CH, CH)], didx[b])

    def start_gather(b):
        pltpu.async_copy(hcat.at[sidx[b]], rows[b], sg[b])

    def wait_gather(b):
        pltpu.make_async_copy(hcat.at[sidx[b]], rows[b], sg[b]).wait()

    def scatter(b):
        pltpu.sync_copy(rows[b], acc.at[didx[b]], add=True)

    load_idx(0, 0)
    start_gather(0)

    def pair(j, _):
        for b in range(2):
            i = 2 * j + b
            load_idx(1 - b, i + 1)
            start_gather(1 - b)

            wait_gather(b)
            scatter(b)
        return ()

    lax.fori_loop(0, (NITER - 1) // 2, pair, (), unroll=False)
    wait_gather(0)
    scatter(0)
    plsc.subcore_barrier()

    @pl.when(s < NS - 1)
    def _():
        pltpu.sync_copy(acc.at[pl.ds(s * ROWS_A, ROWS_A)],
                        agg.at[pl.ds(c * N_NODES + s * ROWS_A, ROWS_A)])

    @pl.when(s == NS - 1)
    def _():
        pltpu.sync_copy(
            acc.at[pl.ds((NS - 1) * ROWS_A, ROWS_LAST)],
            agg.at[pl.ds(c * N_NODES + (NS - 1) * ROWS_A, ROWS_LAST)])


@functools.cache
def _sc_agg():
    return pl.kernel(
        _sc_agg_body,
        out_type=jax.ShapeDtypeStruct((2 * N_NODES, HALF), jnp.float32),
        mesh=plsc.VectorSubcoreMesh(core_axis_name="c", subcore_axis_name="s",
                                    num_cores=NC, num_subcores=NS),
        scratch_types=[
            pltpu.VMEM((CH,), jnp.int32),
            pltpu.VMEM((CH,), jnp.int32),
            pltpu.VMEM((CH,), jnp.int32),
            pltpu.VMEM((CH,), jnp.int32),
            pltpu.VMEM((CH, HALF), jnp.float32),
            pltpu.VMEM((CH, HALF), jnp.float32),
            pltpu.VMEM_SHARED((N_NODES, HALF), jnp.float32),
            pltpu.SemaphoreType.DMA,
            pltpu.SemaphoreType.DMA,
        ],
    )


def _k1_body(eps_ref, hcat_ref, agg_ref, w_ref, b_ref, y_ref, st_ref):
    i = pl.program_id(0)

    @pl.when(i == 0)
    def _():
        st_ref[...] = jnp.zeros_like(st_ref)

    scale = 1.0 + eps_ref[0, 0]
    z = jnp.concatenate(
        [scale * hcat_ref[q] + agg_ref[q] for q in range(2)], axis=1)
    y = jnp.dot(z, w_ref[...], preferred_element_type=jnp.float32) + b_ref[...]
    y_ref[...] = y
    st_ref[...] += jnp.stack([jnp.sum(y, axis=0), jnp.sum(y * y, axis=0)])


def _bn_from_stats(st):
    m = st[0:1, :] / N_NODES
    v = st[1:2, :] / N_NODES - m * m
    return m, lax.rsqrt(v + BN_EPS)


def _k2_body(y_ref, st_ref, g_ref, be_ref, w_ref, b_ref, y2_ref, st2_ref):
    i = pl.program_id(0)

    @pl.when(i == 0)
    def _():
        st2_ref[...] = jnp.zeros_like(st2_ref)

    m, r = _bn_from_stats(st_ref[...])
    yn = jnp.maximum((y_ref[...] - m) * r * g_ref[...] + be_ref[...], 0.0)
    y2 = jnp.dot(yn, w_ref[...], preferred_element_type=jnp.float32) + b_ref[...]
    y2_ref[...] = y2
    st2_ref[...] += jnp.stack([jnp.sum(y2, axis=0), jnp.sum(y2 * y2, axis=0)])


def _k3_body(y_ref, st_ref, g_ref, be_ref, batch_ref, out_ref, pool_ref):
    i = pl.program_id(0)

    @pl.when(i == 0)
    def _():
        pool_ref[...] = jnp.zeros_like(pool_ref)

    m, r = _bn_from_stats(st_ref[...])
    h = jnp.maximum((y_ref[...] - m) * r * g_ref[...] + be_ref[...], 0.0)
    for q in range(2):
        out_ref[q] = h[:, HALF * q:HALF * (q + 1)]
    gids = lax.broadcasted_iota(jnp.int32, (BLK, N_GRAPHS), 1)
    mask = (batch_ref[...] == gids).astype(jnp.float32)
    pool_ref[...] += lax.dot_general(
        mask, h, (((0,), (0,)), ((), ())), preferred_element_type=jnp.float32,
        precision=lax.Precision.HIGHEST)


def _kfin_body(p1_ref, p2_ref, p3_ref, wd1_ref, bd1_ref, wd2_ref, bd2_ref,
               out_ref):
    cat = jnp.concatenate([p1_ref[...], p2_ref[...], p3_ref[...]], axis=1)
    t = jnp.maximum(
        jnp.dot(cat, wd1_ref[...], preferred_element_type=jnp.float32)
        + bd1_ref[...], 0.0)
    out_ref[...] = (jnp.dot(t, wd2_ref[...], preferred_element_type=jnp.float32)
                    + bd2_ref[0, 0])


_row_blk = pl.BlockSpec((BLK, D), lambda i: (i, 0))
_half_blk = pl.BlockSpec((2, BLK, HALF), lambda i: (0, i, 0))
_full_st = pl.BlockSpec((2, D), lambda i: (0, 0))
_full_vec = pl.BlockSpec((1, D), lambda i: (0, 0))
_full_w = pl.BlockSpec((D, D), lambda i: (0, 0))

_k1 = pl.pallas_call(
    _k1_body,
    grid=(GRID,),
    in_specs=[
        pl.BlockSpec(memory_space=pltpu.SMEM),
        _half_blk, _half_blk, _full_w, _full_vec,
    ],
    out_specs=[_row_blk, _full_st],
    out_shape=[
        jax.ShapeDtypeStruct((N_NODES, D), jnp.float32),
        jax.ShapeDtypeStruct((2, D), jnp.float32),
    ],
)

_k2 = pl.pallas_call(
    _k2_body,
    grid=(GRID,),
    in_specs=[_row_blk, _full_st, _full_vec, _full_vec, _full_w, _full_vec],
    out_specs=[_row_blk, _full_st],
    out_shape=[
        jax.ShapeDtypeStruct((N_NODES, D), jnp.float32),
        jax.ShapeDtypeStruct((2, D), jnp.float32),
    ],
)

_k3 = pl.pallas_call(
    _k3_body,
    grid=(GRID,),
    in_specs=[
        _row_blk, _full_st, _full_vec, _full_vec,
        pl.BlockSpec((BLK, 1), lambda i: (i, 0)),
    ],
    out_specs=[_half_blk, pl.BlockSpec((N_GRAPHS, D), lambda i: (0, 0))],
    out_shape=[
        jax.ShapeDtypeStruct((2, N_NODES, HALF), jnp.float32),
        jax.ShapeDtypeStruct((N_GRAPHS, D), jnp.float32),
    ],
)

_kfin = pl.pallas_call(
    _kfin_body,
    in_specs=[
        pl.BlockSpec((N_GRAPHS, D), lambda: (0, 0)),
        pl.BlockSpec((N_GRAPHS, D), lambda: (0, 0)),
        pl.BlockSpec((N_GRAPHS, D), lambda: (0, 0)),
        pl.BlockSpec((L * D, PH), lambda: (0, 0)),
        pl.BlockSpec((1, PH), lambda: (0, 0)),
        pl.BlockSpec((PH, 1), lambda: (0, 0)),
        pl.BlockSpec(memory_space=pltpu.SMEM),
    ],
    out_specs=pl.BlockSpec((N_GRAPHS, 1), lambda: (0, 0)),
    out_shape=jax.ShapeDtypeStruct((N_GRAPHS, 1), jnp.float32),
)


def kernel(x, edge_index, batch, eps, W1, b1, g1, be1, W2, b2, g2, be2,
           Wd1, bd1, Wd2, bd2):
    src = edge_index[0]
    dst = edge_index[1]
    srcx = jnp.concatenate([src, src + N_NODES])
    zeros = jnp.zeros((N_NODES, HALF), jnp.float32)
    batch2 = batch.reshape(N_NODES, 1)

    hcat = jnp.concatenate([x[:, :HALF], x[:, HALF:]], axis=0)

    pools = []
    for l in range(L):
        agg = _sc_agg()(hcat, srcx, dst, zeros)
        eps_l = eps[l].reshape(1, 1)
        h3 = hcat.reshape(2, N_NODES, HALF)
        a3 = agg.reshape(2, N_NODES, HALF)
        y1, st1 = _k1(eps_l, h3, a3, W1[l], b1[l].reshape(1, D))
        y2, st2 = _k2(y1, st1, g1[l].reshape(1, D), be1[l].reshape(1, D),
                      W2[l], b2[l].reshape(1, D))
        hnew, pool = _k3(y2, st2, g2[l].reshape(1, D), be2[l].reshape(1, D),
                         batch2)
        pools.append(pool)
        hcat = hnew.reshape(2 * N_NODES, HALF)

    out = _kfin(pools[0], pools[1], pools[2], Wd1, bd1.reshape(1, PH),
                Wd2, bd2.reshape(1, 1))
    return out
